# baseline (speedup 1.0000x reference)
import functools

import jax
import jax.numpy as jnp
from jax import lax
from jax.experimental import pallas as pl
from jax.experimental.pallas import tpu as pltpu
from jax.experimental.pallas import tpu_sc as plsc

NC = 2
NS = 16
NW = NC * NS
L = 16

B = 16384
D = 16
V = 1000000
ROWPAD = 977 * 1024
VALN = 999936
NTAIL = V - VALN
BPW = B // NW
IDXW = 128
NCHUNK = BPW // IDXW


def _tc_detile(u_ref, i_ref, uo_ref, io_ref, sem):
    copies = []
    for d in range(D):
        copies.append(pltpu.make_async_copy(
            u_ref.at[d, pl.ds(0, VALN)],
            uo_ref.at[pl.ds(d * ROWPAD, VALN)], sem))
        copies.append(pltpu.make_async_copy(
            i_ref.at[d, pl.ds(0, VALN)],
            io_ref.at[pl.ds(d * ROWPAD, VALN)], sem))
    for cp in copies:
        cp.start()
    for cp in copies:
        cp.wait()


def _detile(ut, it):
    return pl.pallas_call(
        _tc_detile,
        in_specs=[
            pl.BlockSpec(memory_space=pl.ANY),
            pl.BlockSpec(memory_space=pl.ANY),
        ],
        out_specs=[
            pl.BlockSpec(memory_space=pl.ANY),
            pl.BlockSpec(memory_space=pl.ANY),
        ],
        out_shape=[
            jax.ShapeDtypeStruct((D * ROWPAD,), jnp.float32),
            jax.ShapeDtypeStruct((D * ROWPAD,), jnp.float32),
        ],
        scratch_shapes=[pltpu.SemaphoreType.DMA],
    )(ut, it)


def _sc_body(uids_hbm, iids_hbm, utab_hbm, itab_hbm, utail_hbm, itail_hbm,
             out_hbm, uidx_v, iidx_v, udata_v, idata_v, utail_v, itail_v,
             out_v, sem):
    wid = lax.axis_index("s") * NC + lax.axis_index("c")

    pltpu.sync_copy(uids_hbm.at[wid], uidx_v)
    pltpu.sync_copy(iids_hbm.at[wid], iidx_v)
    pltpu.sync_copy(utail_hbm, utail_v)
    pltpu.sync_copy(itail_hbm, itail_v)

    copies = []
    for d in range(D):
        u_d = utab_hbm.at[pl.ds(d * ROWPAD, V)]
        i_d = itab_hbm.at[pl.ds(d * ROWPAD, V)]
        for j in range(NCHUNK):
            copies.append(pltpu.async_copy(
                u_d.at[uidx_v.at[j]],
                udata_v.at[d, pl.ds(j * IDXW, IDXW)], sem))
            copies.append(pltpu.async_copy(
                i_d.at[iidx_v.at[j]],
                idata_v.at[d, pl.ds(j * IDXW, IDXW)], sem))
    for cp in copies:
        cp.wait()

    lane_i32 = lax.iota(jnp.int32, L)
    del lane_i32

    def blk(k, carry):
        sl = pl.ds(k * L, L)
        acc = udata_v[0, sl] * idata_v[0, sl]
        for d in range(1, D):
            acc = acc + udata_v[d, sl] * idata_v[d, sl]
        out_v[sl] = acc

        jc = k // 8
        oc = (k % 8) * L
        uvec = uidx_v[jc, pl.ds(oc, L)]
        ivec = iidx_v[jc, pl.ds(oc, L)]
        mu = uvec >= VALN
        mi = ivec >= VALN
        anyfix = jnp.any(jnp.logical_or(mu, mi))

        @pl.when(anyfix)
        def _fix():
            ut_base = (uvec - VALN) * D
            it_base = (ivec - VALN) * D
            acc2 = jnp.zeros((L,), jnp.float32)
            for d in range(D):
                ufix = plsc.load_gather(utail_v, [ut_base + d], mask=mu)
                ifix = plsc.load_gather(itail_v, [it_base + d], mask=mi)
                ud = jnp.where(mu, ufix, udata_v[d, sl])
                idv = jnp.where(mi, ifix, idata_v[d, sl])
                acc2 = acc2 + ud * idv
            out_v[sl] = acc2

        return carry

    lax.fori_loop(0, BPW // L, blk, 0)
    pltpu.sync_copy(out_v, out_hbm.at[wid])


def kernel(user_ids, item_ids, user_table, item_table):
    mesh = plsc.VectorSubcoreMesh(core_axis_name="c", subcore_axis_name="s")

    sc_call = functools.partial(
        pl.kernel,
        out_type=jax.ShapeDtypeStruct((NW, BPW), jnp.float32),
        mesh=mesh,
        scratch_types=[
            pltpu.VMEM((NCHUNK, IDXW), jnp.int32),
            pltpu.VMEM((NCHUNK, IDXW), jnp.int32),
            pltpu.VMEM((D, BPW), jnp.float32),
            pltpu.VMEM((D, BPW), jnp.float32),
            pltpu.VMEM((NTAIL * D,), jnp.float32),
            pltpu.VMEM((NTAIL * D,), jnp.float32),
            pltpu.VMEM((BPW,), jnp.float32),
            pltpu.SemaphoreType.DMA,
        ],
        compiler_params=pltpu.CompilerParams(
            needs_layout_passes=False, use_tc_tiling_on_sc=False),
    )(_sc_body)

    uflat, iflat = _detile(user_table.T, item_table.T)
    utail = user_table[VALN:].reshape(NTAIL * D)
    itail = item_table[VALN:].reshape(NTAIL * D)
    uids = user_ids.astype(jnp.int32).reshape(NW, NCHUNK, IDXW)
    iids = item_ids.astype(jnp.int32).reshape(NW, NCHUNK, IDXW)
    out = sc_call(uids, iids, uflat, iflat, utail, itail)
    return out.reshape(B)


# speedup vs baseline: 25.7692x; 25.7692x over previous
import functools

import jax
import jax.numpy as jnp
from jax import lax
from jax.experimental import pallas as pl
from jax.experimental.pallas import tpu as pltpu
from jax.experimental.pallas import tpu_sc as plsc

NC = 2
NS = 16
NW = NC * NS
L = 16

B = 16384
D = 16
V = 1000000
ROWPAD = 977 * 1024
VALN = 999936
NTAIL = V - VALN
BPW = B // NW
IDXW = 128
NCHUNK = BPW // IDXW


CWT = 166656
NSLB = VALN // CWT


def _tc_detile(u_ref, i_ref, uo_ref, io_ref, b0, b1, semr, semw):
    slabs = [(tab, g, k)
             for tab in (0, 1) for g in (0, 1) for k in range(NSLB)]
    bufs = (b0, b1)
    pending = {}
    for idx, (tab, g, k) in enumerate(slabs):
        src_ref = u_ref if tab == 0 else i_ref
        dst_ref = uo_ref if tab == 0 else io_ref
        buf = bufs[idx % 2]
        if idx >= 2:
            for w in pending.pop(idx - 2):
                w.wait()
        rd = pltpu.make_async_copy(
            src_ref.at[pl.ds(8 * g, 8), pl.ds(k * CWT, CWT)], buf, semr)
        rd.start()
        rd.wait()
        ws = []
        for s in range(8):
            w = pltpu.make_async_copy(
                buf.at[s],
                dst_ref.at[pl.ds((8 * g + s) * ROWPAD + k * CWT, CWT)],
                semw)
            w.start()
            ws.append(w)
        pending[idx] = ws
    for ws in pending.values():
        for w in ws:
            w.wait()


def _detile(ut, it):
    return pl.pallas_call(
        _tc_detile,
        in_specs=[
            pl.BlockSpec(memory_space=pl.ANY),
            pl.BlockSpec(memory_space=pl.ANY),
        ],
        out_specs=[
            pl.BlockSpec(memory_space=pl.ANY),
            pl.BlockSpec(memory_space=pl.ANY),
        ],
        out_shape=[
            jax.ShapeDtypeStruct((D * ROWPAD,), jnp.float32),
            jax.ShapeDtypeStruct((D * ROWPAD,), jnp.float32),
        ],
        scratch_shapes=[
            pltpu.VMEM((8, CWT), jnp.float32),
            pltpu.VMEM((8, CWT), jnp.float32),
            pltpu.SemaphoreType.DMA,
            pltpu.SemaphoreType.DMA,
        ],
    )(ut, it)


def _sc_body(uids_hbm, iids_hbm, utab_hbm, itab_hbm, utail_hbm, itail_hbm,
             out_hbm, uidx_v, iidx_v, udata_v, idata_v, utail_v, itail_v,
             out_v, sem):
    wid = lax.axis_index("s") * NC + lax.axis_index("c")

    pltpu.sync_copy(uids_hbm.at[wid], uidx_v)
    pltpu.sync_copy(iids_hbm.at[wid], iidx_v)
    pltpu.sync_copy(utail_hbm, utail_v)
    pltpu.sync_copy(itail_hbm, itail_v)

    copies = []
    for d in range(D):
        u_d = utab_hbm.at[pl.ds(d * ROWPAD, V)]
        i_d = itab_hbm.at[pl.ds(d * ROWPAD, V)]
        for j in range(NCHUNK):
            copies.append(pltpu.async_copy(
                u_d.at[uidx_v.at[j]],
                udata_v.at[d, pl.ds(j * IDXW, IDXW)], sem))
            copies.append(pltpu.async_copy(
                i_d.at[iidx_v.at[j]],
                idata_v.at[d, pl.ds(j * IDXW, IDXW)], sem))
    for cp in copies:
        cp.wait()

    lane_i32 = lax.iota(jnp.int32, L)
    del lane_i32

    def blk(k, carry):
        sl = pl.ds(k * L, L)
        acc = udata_v[0, sl] * idata_v[0, sl]
        for d in range(1, D):
            acc = acc + udata_v[d, sl] * idata_v[d, sl]
        out_v[sl] = acc

        jc = k // 8
        oc = (k % 8) * L
        uvec = uidx_v[jc, pl.ds(oc, L)]
        ivec = iidx_v[jc, pl.ds(oc, L)]
        mu = uvec >= VALN
        mi = ivec >= VALN
        anyfix = jnp.any(jnp.logical_or(mu, mi))

        @pl.when(anyfix)
        def _fix():
            ut_base = (uvec - VALN) * D
            it_base = (ivec - VALN) * D
            acc2 = jnp.zeros((L,), jnp.float32)
            for d in range(D):
                ufix = plsc.load_gather(utail_v, [ut_base + d], mask=mu)
                ifix = plsc.load_gather(itail_v, [it_base + d], mask=mi)
                ud = jnp.where(mu, ufix, udata_v[d, sl])
                idv = jnp.where(mi, ifix, idata_v[d, sl])
                acc2 = acc2 + ud * idv
            out_v[sl] = acc2

        return carry

    lax.fori_loop(0, BPW // L, blk, 0)
    pltpu.sync_copy(out_v, out_hbm.at[wid])


def kernel(user_ids, item_ids, user_table, item_table):
    mesh = plsc.VectorSubcoreMesh(core_axis_name="c", subcore_axis_name="s")

    sc_call = functools.partial(
        pl.kernel,
        out_type=jax.ShapeDtypeStruct((NW, BPW), jnp.float32),
        mesh=mesh,
        scratch_types=[
            pltpu.VMEM((NCHUNK, IDXW), jnp.int32),
            pltpu.VMEM((NCHUNK, IDXW), jnp.int32),
            pltpu.VMEM((D, BPW), jnp.float32),
            pltpu.VMEM((D, BPW), jnp.float32),
            pltpu.VMEM((NTAIL * D,), jnp.float32),
            pltpu.VMEM((NTAIL * D,), jnp.float32),
            pltpu.VMEM((BPW,), jnp.float32),
            pltpu.SemaphoreType.DMA,
        ],
        compiler_params=pltpu.CompilerParams(
            needs_layout_passes=False, use_tc_tiling_on_sc=False),
    )(_sc_body)

    uflat, iflat = _detile(user_table.T, item_table.T)
    utail = user_table[VALN:].reshape(NTAIL * D)
    itail = item_table[VALN:].reshape(NTAIL * D)
    uids = user_ids.astype(jnp.int32).reshape(NW, NCHUNK, IDXW)
    iids = item_ids.astype(jnp.int32).reshape(NW, NCHUNK, IDXW)
    out = sc_call(uids, iids, uflat, iflat, utail, itail)
    return out.reshape(B)
